# baseline (speedup 1.0000x reference)
import functools

import jax
import jax.numpy as jnp
from jax import lax
from jax.experimental import pallas as pl
from jax.experimental.pallas import tpu as pltpu
from jax.experimental.pallas import tpu_sc as plsc

_NC = 2
_NS = 16
_NW = _NC * _NS
_LANES = 16

_R_SC = 512
_BR = 512


def _sc_body(sa_tab_hbm, som_tab_hbm, trep_hbm, x_hbm, n_hbm, out_hbm,
             tr_a, tr_b, sa_rep, som_rep, xb, nb, ob,
             sem_a, sem_b, *, rpw, c, l):
    wid = lax.axis_index("s") * _NC + lax.axis_index("c")
    g0 = wid * rpw
    nrep = rpw * _LANES
    pltpu.sync_copy(trep_hbm.at[pl.ds(wid * nrep, 128)], tr_a)
    pltpu.sync_copy(trep_hbm.at[pl.ds(wid * nrep + 128, 128)], tr_b)
    cp1 = pltpu.async_copy(sa_tab_hbm.at[tr_a], sa_rep.at[pl.ds(0, 128)], sem_a)
    cp2 = pltpu.async_copy(sa_tab_hbm.at[tr_b], sa_rep.at[pl.ds(128, 128)], sem_a)
    cp3 = pltpu.async_copy(som_tab_hbm.at[tr_a], som_rep.at[pl.ds(0, 128)], sem_b)
    cp4 = pltpu.async_copy(som_tab_hbm.at[tr_b], som_rep.at[pl.ds(128, 128)], sem_b)
    cpx = pltpu.async_copy(x_hbm.at[pl.ds(g0, rpw)], xb, sem_a)
    cpn = pltpu.async_copy(n_hbm.at[pl.ds(g0, rpw)], nb, sem_b)
    cp1.wait()
    cp2.wait()
    cp3.wait()
    cp4.wait()
    cpx.wait()
    cpn.wait()
    nvec = l // _LANES
    for r in range(rpw):
        sa16 = sa_rep[pl.ds(r * _LANES, _LANES)]
        som16 = som_rep[pl.ds(r * _LANES, _LANES)]
        for ci in range(c):
            def body(j, carry):
                xv = xb[r, ci, pl.ds(j * _LANES, _LANES)]
                nv = nb[r, ci, pl.ds(j * _LANES, _LANES)]
                ob[r, ci, pl.ds(j * _LANES, _LANES)] = sa16 * xv + som16 * nv
                return carry
            lax.fori_loop(0, nvec, body, 0, unroll=4)
    pltpu.sync_copy(ob, out_hbm.at[pl.ds(g0, rpw)])


def _sc_combine(sa_tab, som_tab, t_rep, x_0, noise):
    b, c, l = x_0.shape
    rpw = _R_SC // _NW
    mesh = plsc.VectorSubcoreMesh(core_axis_name="c", subcore_axis_name="s")
    body = functools.partial(_sc_body, rpw=rpw, c=c, l=l)
    k = pl.kernel(
        body,
        out_type=jax.ShapeDtypeStruct((_R_SC, c, l), jnp.float32),
        mesh=mesh,
        scratch_types=[
            pltpu.VMEM((128,), jnp.int32),
            pltpu.VMEM((128,), jnp.int32),
            pltpu.VMEM((rpw * _LANES,), jnp.float32),
            pltpu.VMEM((rpw * _LANES,), jnp.float32),
            pltpu.VMEM((rpw, c, l), jnp.float32),
            pltpu.VMEM((rpw, c, l), jnp.float32),
            pltpu.VMEM((rpw, c, l), jnp.float32),
            pltpu.SemaphoreType.DMA,
            pltpu.SemaphoreType.DMA,
        ],
    )
    return k(sa_tab, som_tab, t_rep, x_0, noise)


def _tc_body(t_ref, sa_tab_ref, som_tab_ref, x_ref, n_ref, o_ref):
    br = t_ref.shape[0]
    tt = sa_tab_ref.shape[1]
    tcol = t_ref[...][:, None]
    iota = lax.broadcasted_iota(jnp.int32, (br, tt), 1)
    onehot = (iota == tcol)
    sa = jnp.sum(jnp.where(onehot, sa_tab_ref[...], 0.0), axis=1)[:, None, None]
    som = jnp.sum(jnp.where(onehot, som_tab_ref[...], 0.0), axis=1)[:, None, None]
    o_ref[...] = sa * x_ref[...] + som * n_ref[...]


def _tc_combine(t, sa_tab2, som_tab2, x_0, noise):
    b, c, l = x_0.shape
    tt = sa_tab2.shape[1]
    skip = _R_SC // _BR
    grid = ((b - _R_SC) // _BR,)
    row_spec = pl.BlockSpec((_BR, c, l), lambda i: (i + skip, 0, 0))
    t_spec = pl.BlockSpec((_BR,), lambda i: (i + skip,))
    tab_spec = pl.BlockSpec((1, tt), lambda i: (0, 0))
    return pl.pallas_call(
        _tc_body,
        grid=grid,
        in_specs=[t_spec, tab_spec, tab_spec, row_spec, row_spec],
        out_specs=row_spec,
        out_shape=jax.ShapeDtypeStruct((b, c, l), jnp.float32),
    )(t, sa_tab2, som_tab2, x_0, noise)


def kernel(x_0, t, noise, sqrt_alphas_cumprod, sqrt_one_minus_alphas_cumprod):
    tt = sqrt_alphas_cumprod.shape[0]
    t_rep = jnp.repeat(t[:_R_SC], _LANES)
    sc_out = _sc_combine(sqrt_alphas_cumprod, sqrt_one_minus_alphas_cumprod,
                         t_rep, x_0, noise)
    tc_out = _tc_combine(t, sqrt_alphas_cumprod.reshape(1, tt),
                         sqrt_one_minus_alphas_cumprod.reshape(1, tt),
                         x_0, noise)
    return lax.dynamic_update_slice(tc_out, sc_out, (0, 0, 0))


# speedup vs baseline: 1.0633x; 1.0633x over previous
import functools

import jax
import jax.numpy as jnp
from jax import lax
from jax.experimental import pallas as pl
from jax.experimental.pallas import tpu as pltpu
from jax.experimental.pallas import tpu_sc as plsc

_NC = 2
_NS = 16
_NW = _NC * _NS
_LANES = 16

_R_SC = 256
_BR = 256


def _sc_body(sa_tab_hbm, som_tab_hbm, trep_hbm, x_hbm, n_hbm, out_hbm,
             tr_a, tr_b, sa_rep, som_rep, xb, nb, ob,
             sem_a, sem_b, *, rpw, c, l):
    wid = lax.axis_index("s") * _NC + lax.axis_index("c")
    g0 = wid * rpw
    nrep = rpw * _LANES
    trs = (tr_a, tr_b)
    waits = []
    for k in range(nrep // 128):
        pltpu.sync_copy(trep_hbm.at[pl.ds(wid * nrep + k * 128, 128)], trs[k])
        waits.append(pltpu.async_copy(
            sa_tab_hbm.at[trs[k]], sa_rep.at[pl.ds(k * 128, 128)], sem_a))
        waits.append(pltpu.async_copy(
            som_tab_hbm.at[trs[k]], som_rep.at[pl.ds(k * 128, 128)], sem_b))
    waits.append(pltpu.async_copy(x_hbm.at[pl.ds(g0, rpw)], xb, sem_a))
    waits.append(pltpu.async_copy(n_hbm.at[pl.ds(g0, rpw)], nb, sem_b))
    for w in waits:
        w.wait()
    nvec = l // _LANES
    for r in range(rpw):
        sa16 = sa_rep[pl.ds(r * _LANES, _LANES)]
        som16 = som_rep[pl.ds(r * _LANES, _LANES)]
        for ci in range(c):
            def body(j, carry):
                xv = xb[r, ci, pl.ds(j * _LANES, _LANES)]
                nv = nb[r, ci, pl.ds(j * _LANES, _LANES)]
                ob[r, ci, pl.ds(j * _LANES, _LANES)] = sa16 * xv + som16 * nv
                return carry
            lax.fori_loop(0, nvec, body, 0, unroll=8)
    pltpu.sync_copy(ob, out_hbm.at[pl.ds(g0, rpw)])


def _sc_combine(sa_tab, som_tab, t_rep, x_0, noise):
    b, c, l = x_0.shape
    rpw = _R_SC // _NW
    mesh = plsc.VectorSubcoreMesh(core_axis_name="c", subcore_axis_name="s")
    body = functools.partial(_sc_body, rpw=rpw, c=c, l=l)
    k = pl.kernel(
        body,
        out_type=jax.ShapeDtypeStruct((_R_SC, c, l), jnp.float32),
        mesh=mesh,
        scratch_types=[
            pltpu.VMEM((128,), jnp.int32),
            pltpu.VMEM((128,), jnp.int32),
            pltpu.VMEM((rpw * _LANES,), jnp.float32),
            pltpu.VMEM((rpw * _LANES,), jnp.float32),
            pltpu.VMEM((rpw, c, l), jnp.float32),
            pltpu.VMEM((rpw, c, l), jnp.float32),
            pltpu.VMEM((rpw, c, l), jnp.float32),
            pltpu.SemaphoreType.DMA,
            pltpu.SemaphoreType.DMA,
        ],
    )
    return k(sa_tab, som_tab, t_rep, x_0, noise)


def _tc_body(t_ref, sa_tab_ref, som_tab_ref, x_ref, n_ref, o_ref):
    br = t_ref.shape[0]
    tt = sa_tab_ref.shape[1]
    tcol = t_ref[...][:, None]
    iota = lax.broadcasted_iota(jnp.int32, (br, tt), 1)
    onehot = (iota == tcol)
    sa = jnp.sum(jnp.where(onehot, sa_tab_ref[...], 0.0), axis=1)[:, None, None]
    som = jnp.sum(jnp.where(onehot, som_tab_ref[...], 0.0), axis=1)[:, None, None]
    o_ref[...] = sa * x_ref[...] + som * n_ref[...]


def _tc_combine(t, sa_tab2, som_tab2, x_0, noise):
    b, c, l = x_0.shape
    tt = sa_tab2.shape[1]
    skip = _R_SC // _BR
    grid = ((b - _R_SC) // _BR,)
    row_spec = pl.BlockSpec((_BR, c, l), lambda i: (i + skip, 0, 0))
    t_spec = pl.BlockSpec((_BR,), lambda i: (i + skip,))
    tab_spec = pl.BlockSpec((1, tt), lambda i: (0, 0))
    return pl.pallas_call(
        _tc_body,
        grid=grid,
        in_specs=[t_spec, tab_spec, tab_spec, row_spec, row_spec],
        out_specs=row_spec,
        out_shape=jax.ShapeDtypeStruct((b, c, l), jnp.float32),
    )(t, sa_tab2, som_tab2, x_0, noise)


def kernel(x_0, t, noise, sqrt_alphas_cumprod, sqrt_one_minus_alphas_cumprod):
    tt = sqrt_alphas_cumprod.shape[0]
    t_rep = jnp.repeat(t[:_R_SC], _LANES)
    sc_out = _sc_combine(sqrt_alphas_cumprod, sqrt_one_minus_alphas_cumprod,
                         t_rep, x_0, noise)
    tc_out = _tc_combine(t, sqrt_alphas_cumprod.reshape(1, tt),
                         sqrt_one_minus_alphas_cumprod.reshape(1, tt),
                         x_0, noise)
    return lax.dynamic_update_slice(tc_out, sc_out, (0, 0, 0))


# speedup vs baseline: 1.1498x; 1.0814x over previous
import functools

import jax
import jax.numpy as jnp
from jax import lax
from jax.experimental import pallas as pl
from jax.experimental.pallas import tpu as pltpu
from jax.experimental.pallas import tpu_sc as plsc

_NC = 2
_NS = 16
_NW = _NC * _NS


def _sc_gather_body(sa_tab_hbm, som_tab_hbm, t_hbm, sa_out_hbm, som_out_hbm,
                    t_v, sa_o_v, som_o_v, sem_a, sem_b, *, b_per_w):
    wid = lax.axis_index("s") * _NC + lax.axis_index("c")
    base = wid * b_per_w
    pltpu.sync_copy(t_hbm.at[pl.ds(base, b_per_w)], t_v)
    cp_a = pltpu.async_copy(sa_tab_hbm.at[t_v], sa_o_v, sem_a)
    cp_b = pltpu.async_copy(som_tab_hbm.at[t_v], som_o_v, sem_b)
    cp_a.wait()
    cp_b.wait()
    pltpu.sync_copy(sa_o_v, sa_out_hbm.at[pl.ds(base, b_per_w)])
    pltpu.sync_copy(som_o_v, som_out_hbm.at[pl.ds(base, b_per_w)])


def _sc_gather(sa_tab, som_tab, t):
    b = t.shape[0]
    b_per_w = b // _NW
    mesh = plsc.VectorSubcoreMesh(core_axis_name="c", subcore_axis_name="s")
    body = functools.partial(_sc_gather_body, b_per_w=b_per_w)
    k = pl.kernel(
        body,
        out_type=(
            jax.ShapeDtypeStruct((b,), jnp.float32),
            jax.ShapeDtypeStruct((b,), jnp.float32),
        ),
        mesh=mesh,
        scratch_types=[
            pltpu.VMEM((b_per_w,), jnp.int32),
            pltpu.VMEM((b_per_w,), jnp.float32),
            pltpu.VMEM((b_per_w,), jnp.float32),
            pltpu.SemaphoreType.DMA,
            pltpu.SemaphoreType.DMA,
        ],
    )
    return k(sa_tab, som_tab, t)


def _combine_body(sa_ref, som_ref, x_ref, n_ref, o_ref):
    sa = sa_ref[...][:, None, None]
    som = som_ref[...][:, None, None]
    o_ref[...] = sa * x_ref[...] + som * n_ref[...]


def _combine(sa_b, som_b, x, n, block_rows):
    b, c, l = x.shape
    grid = (b // block_rows,)
    row_spec = pl.BlockSpec((block_rows, c, l), lambda i: (i, 0, 0))
    coef_spec = pl.BlockSpec((block_rows,), lambda i: (i,))
    return pl.pallas_call(
        _combine_body,
        grid=grid,
        in_specs=[coef_spec, coef_spec, row_spec, row_spec],
        out_specs=row_spec,
        out_shape=jax.ShapeDtypeStruct((b, c, l), jnp.float32),
    )(sa_b, som_b, x, n)


def kernel(x_0, t, noise, sqrt_alphas_cumprod, sqrt_one_minus_alphas_cumprod):
    sa_b, som_b = _sc_gather(sqrt_alphas_cumprod,
                             sqrt_one_minus_alphas_cumprod, t)
    return _combine(sa_b, som_b, x_0, noise, 512)
